# baseline (speedup 1.0000x reference)
import functools

import jax
import jax.numpy as jnp
from jax import lax
from jax.experimental import pallas as pl
from jax.experimental.pallas import tpu as pltpu
from jax.experimental.pallas import tpu_sc as plsc

V = 100000
P = 128
NB = 4096
S = 50
NW = 32
PPT = P // NW
L = 16
UNROLL = 16
NI = 4
NO = 2


def _make_gather():
    mesh = plsc.VectorSubcoreMesh(core_axis_name="c", subcore_axis_name="s")

    @functools.partial(
        pl.kernel,
        mesh=mesh,
        compiler_params=pltpu.CompilerParams(needs_layout_passes=False),
        out_type=jax.ShapeDtypeStruct((S * P, NB), jnp.float32),
        scratch_types=[
            pltpu.VMEM((V,), jnp.float32),
        ]
        + [pltpu.VMEM((NB,), jnp.int32)] * NI
        + [pltpu.VMEM((NB,), jnp.float32)] * NO
        + [pltpu.SemaphoreType.DMA] * (NI + NO),
    )
    def gather_kernel(xT, idxT, outT, plane, *scr):
        ibuf = scr[:NI]
        obuf = scr[NI:NI + NO]
        isem = scr[NI + NO:2 * NI + NO]
        osem = scr[2 * NI + NO:]
        wid = lax.axis_index("s") * 2 + lax.axis_index("c")

        def gather_row(idxb, outb):
            @plsc.parallel_loop(0, NB, L, unroll=UNROLL)
            def body(i):
                ids = idxb[pl.ds(i, L)]
                outb[pl.ds(i, L)] = plsc.load_gather(plane, [ids])

        def wait_idx(u):
            pltpu.make_async_copy(idxT.at[0], ibuf[u], isem[u]).wait()

        def wait_out(v):
            pltpu.make_async_copy(obuf[v], outT.at[0], osem[v]).wait()

        def step(s, u, p, prefetch, outwait):
            wait_idx(u)
            if outwait:
                wait_out(u % NO)
            gather_row(ibuf[u], obuf[u % NO])
            pltpu.async_copy(obuf[u % NO], outT.at[s * P + p], osem[u % NO])
            if prefetch:
                pltpu.async_copy(idxT.at[s + NI], ibuf[u], isem[u])

        for pi in range(PPT):
            p = wid * PPT + pi
            pltpu.sync_copy(xT.at[p], plane)
            for u in range(NI):
                pltpu.async_copy(idxT.at[u], ibuf[u], isem[u])

            for u in range(NI):
                step(u, u, p, prefetch=True, outwait=(u >= NO))

            def quad(g, carry):
                s0 = 4 * g
                for u in range(NI):
                    step(s0 + u, u, p, prefetch=True, outwait=True)
                return carry

            lax.fori_loop(1, (S - NI - 2) // NI, quad, 0)

            for u in range(NI):
                step(S - 6 + u, u, p, prefetch=(u < 2), outwait=True)
            for u in range(2):
                step(S - 2 + u, u, p, prefetch=False, outwait=True)

            wait_out(0)
            wait_out(1)

    return gather_kernel


_gather = _make_gather()


@jax.jit
def kernel(x, index):
    b, s = index.shape
    xT = x.transpose(1, 2, 0).reshape(P, V)
    idxT = index.astype(jnp.int32).T
    outT = _gather(xT, idxT)
    return outT.reshape(s, 16, 8, b).transpose(3, 0, 1, 2)
